# baseline (speedup 1.0000x reference)
import functools

import jax
import jax.numpy as jnp
from jax import lax
from jax.experimental import pallas as pl
from jax.experimental.pallas import tpu as pltpu
from jax.experimental.pallas import tpu_sc as plsc

N = 10000
E = 320000
D_IN = 128
H = 32
W2P = 16

NC, NS = 2, 16
NT = NC * NS
CHUNK = 256
NCHUNK = 40
NBUF = 4
PER_TILE = NCHUNK * CHUNK
EPAD = NT * PER_TILE
NP = 10240
ROWS_PER_TILE = NP // NS

_mesh = plsc.VectorSubcoreMesh(core_axis_name="c", subcore_axis_name="s")


def _make_edge_pass(width):
  @functools.partial(
      pl.kernel,
      out_type=jax.ShapeDtypeStruct((NC, NP, width), jnp.float32),
      mesh=_mesh,
      compiler_params=pltpu.CompilerParams(use_tc_tiling_on_sc=False),
      scratch_types=[
          pltpu.VMEM((NCHUNK, CHUNK), jnp.int32),
          pltpu.VMEM((NCHUNK, CHUNK), jnp.int32),
          pltpu.VMEM((NBUF, CHUNK, width), jnp.float32),
          pltpu.VMEM_SHARED((NP, width), jnp.float32),
          [pltpu.SemaphoreType.DMA] * NBUF,
          [pltpu.SemaphoreType.DMA] * NBUF,
      ],
  )
  def edge_pass(src_hbm, dst_hbm, y_hbm, out_hbm,
                src_v, dst_v, vals_v, agg_s, gsem, ssem):
    c = lax.axis_index("c")
    s = lax.axis_index("s")
    wid = c * NS + s

    @pl.loop(0, CHUNK)
    def _(r):
      for k in range(width // 16):
        vals_v.at[0, r, pl.ds(k * 16, 16)][...] = jnp.zeros((16,), jnp.float32)

    row0 = s * ROWS_PER_TILE
    nfull, rem = divmod(ROWS_PER_TILE, CHUNK)
    @pl.loop(0, nfull)
    def _(i):
      pltpu.sync_copy(vals_v.at[0], agg_s.at[pl.ds(row0 + i * CHUNK, CHUNK)])
    if rem:
      pltpu.sync_copy(vals_v.at[0].at[pl.ds(0, rem)],
                      agg_s.at[pl.ds(row0 + nfull * CHUNK, rem)])

    pltpu.sync_copy(src_hbm.at[wid], src_v)
    pltpu.sync_copy(dst_hbm.at[wid], dst_v)

    plsc.subcore_barrier()

    def start_gather(j, b):
      pltpu.async_copy(y_hbm.at[src_v.at[j]], vals_v.at[b], gsem[b])

    def wait_gather(j, b):
      pltpu.make_async_copy(y_hbm.at[src_v.at[j]], vals_v.at[b],
                            gsem[b]).wait()

    def start_scatter(j, b):
      pltpu.async_copy(vals_v.at[b], agg_s.at[dst_v.at[j]], ssem[b],
                       add=True)

    def wait_scatter(j, b):
      pltpu.make_async_copy(vals_v.at[b], agg_s.at[dst_v.at[j]],
                            ssem[b]).wait()

    for b in range(NBUF):
      start_gather(b, b)

    @pl.loop(0, NCHUNK - NBUF, step=NBUF)
    def _(j):
      for b in range(NBUF):
        wait_gather(j + b, b)
        start_scatter(j + b, b)
      for b in range(NBUF):
        wait_scatter(j + b, b)
        start_gather(j + b + NBUF, b)

    last = NCHUNK - NBUF
    for b in range(NBUF):
      wait_gather(last + b, b)
      start_scatter(last + b, b)
    for b in range(NBUF):
      wait_scatter(last + b, b)

    plsc.subcore_barrier()

    @pl.loop(0, nfull)
    def _(i):
      r = row0 + i * CHUNK
      pltpu.sync_copy(agg_s.at[pl.ds(r, CHUNK)],
                      out_hbm.at[c].at[pl.ds(r, CHUNK)])
    if rem:
      r = row0 + nfull * CHUNK
      pltpu.sync_copy(agg_s.at[pl.ds(r, rem)],
                      out_hbm.at[c].at[pl.ds(r, rem)])

  return edge_pass


_edge_pass_l1 = _make_edge_pass(H)
_edge_pass_l2 = _make_edge_pass(W2P)


@functools.partial(
    pl.kernel,
    out_type=jax.ShapeDtypeStruct((NC, NP), jnp.float32),
    mesh=_mesh,
    compiler_params=pltpu.CompilerParams(use_tc_tiling_on_sc=False),
    scratch_types=[
        pltpu.VMEM((NCHUNK, CHUNK), jnp.int32),
        pltpu.VMEM((CHUNK,), jnp.float32),
        pltpu.VMEM((ROWS_PER_TILE,), jnp.float32),
        pltpu.VMEM_SHARED((NP,), jnp.float32),
    ],
)
def _deg_pass(dst_hbm, out_hbm, dst_v, ones_v, zeros_v, deg_s):
  c = lax.axis_index("c")
  s = lax.axis_index("s")
  wid = c * NS + s

  for k in range(CHUNK // 16):
    ones_v.at[pl.ds(k * 16, 16)][...] = jnp.ones((16,), jnp.float32)

  @pl.loop(0, ROWS_PER_TILE // 16)
  def _(k):
    zeros_v.at[pl.ds(k * 16, 16)][...] = jnp.zeros((16,), jnp.float32)

  row0 = s * ROWS_PER_TILE
  pltpu.sync_copy(zeros_v, deg_s.at[pl.ds(row0, ROWS_PER_TILE)])
  pltpu.sync_copy(dst_hbm.at[wid], dst_v)

  plsc.subcore_barrier()

  @pl.loop(0, NCHUNK)
  def _(j):
    pltpu.sync_copy(ones_v, deg_s.at[dst_v.at[j]], add=True)

  plsc.subcore_barrier()

  pltpu.sync_copy(deg_s.at[pl.ds(row0, ROWS_PER_TILE)],
                  out_hbm.at[c].at[pl.ds(row0, ROWS_PER_TILE)])


def _tc_xw(x_ref, w_ref, o_ref):
  o_ref[...] = jnp.dot(x_ref[...], w_ref[...],
                       preferred_element_type=jnp.float32)


def _tc_scale(degp_ref, xw_ref, dis_ref, y_ref):
  deg = degp_ref[0] + degp_ref[1] + 1.0
  dis = lax.rsqrt(deg)[:, None]
  dis_ref[...] = dis
  y_ref[...] = xw_ref[...] * dis


def _tc_mid(p_ref, y_ref, dis_ref, b1_ref, w2_ref, z_ref):
  dis = dis_ref[...]
  h = dis * (p_ref[0] + p_ref[1] + y_ref[...]) + b1_ref[...]
  h = jnp.maximum(h, 0.0)
  z_ref[...] = dis * jnp.dot(h, w2_ref[...],
                             preferred_element_type=jnp.float32)


def _tc_final(q_ref, z_ref, dis_ref, b2_ref, o_ref):
  out = dis_ref[...] * (q_ref[0] + q_ref[1] + z_ref[...])
  o_ref[...] = out[:N, :2] + b2_ref[...]


def kernel(x, edge_index, W1, b1, W2, b2):
  ei = edge_index.astype(jnp.int32)
  pad = jnp.full((EPAD - E,), N, jnp.int32)
  srcp = jnp.concatenate([ei[0], pad]).reshape(NT, NCHUNK, CHUNK)
  dstp = jnp.concatenate([ei[1], pad]).reshape(NT, NCHUNK, CHUNK)
  x_pad = jnp.pad(x, ((0, NP - N), (0, 0)))
  w2_pad = jnp.pad(W2, ((0, 0), (0, W2P - 2)))

  degp = _deg_pass(dstp)

  xw = pl.pallas_call(
      _tc_xw,
      out_shape=jax.ShapeDtypeStruct((NP, H), jnp.float32),
  )(x_pad, W1)

  dis, y = pl.pallas_call(
      _tc_scale,
      out_shape=(jax.ShapeDtypeStruct((NP, 1), jnp.float32),
                 jax.ShapeDtypeStruct((NP, H), jnp.float32)),
  )(degp, xw)

  p = _edge_pass_l1(srcp, dstp, y)

  z = pl.pallas_call(
      _tc_mid,
      out_shape=jax.ShapeDtypeStruct((NP, W2P), jnp.float32),
  )(p, y, dis, b1.reshape(1, H), w2_pad)

  q = _edge_pass_l2(srcp, dstp, z)

  out = pl.pallas_call(
      _tc_final,
      out_shape=jax.ShapeDtypeStruct((N, 2), jnp.float32),
  )(q, z, dis, b2.reshape(1, 2))

  return out


# speedup vs baseline: 1.5532x; 1.5532x over previous
import functools

import jax
import jax.numpy as jnp
from jax import lax
from jax.experimental import pallas as pl
from jax.experimental.pallas import tpu as pltpu
from jax.experimental.pallas import tpu_sc as plsc

N = 10000
E = 320000
D_IN = 128
H = 32
W2P = 16

NC, NS = 2, 16
NT = NC * NS
CHUNK = 256
NCHUNK = 40
NBUF = 4
PER_TILE = NCHUNK * CHUNK
EPAD = NT * PER_TILE
NP = 10240
ROWS_PER_TILE = NP // NS

_mesh = plsc.VectorSubcoreMesh(core_axis_name="c", subcore_axis_name="s")


def _make_edge_pass(width):
  @functools.partial(
      pl.kernel,
      out_type=jax.ShapeDtypeStruct((NC, NP, width), jnp.float32),
      mesh=_mesh,
      compiler_params=pltpu.CompilerParams(use_tc_tiling_on_sc=False),
      scratch_types=[
          pltpu.VMEM((NCHUNK, CHUNK), jnp.int32),
          pltpu.VMEM((NCHUNK, CHUNK), jnp.int32),
          pltpu.VMEM((NBUF, CHUNK, width), jnp.float32),
          pltpu.VMEM_SHARED((NP, width), jnp.float32),
          pltpu.VMEM_SHARED((NP, width), jnp.float32),
          [pltpu.SemaphoreType.DMA] * NBUF,
          [pltpu.SemaphoreType.DMA] * NBUF,
      ],
  )
  def edge_pass(src_hbm, dst_hbm, y_hbm, out_hbm,
                src_v, dst_v, vals_v, agg_s, y_s, gsem, ssem):
    c = lax.axis_index("c")
    s = lax.axis_index("s")
    wid = c * NS + s

    @pl.loop(0, CHUNK)
    def _(r):
      for k in range(width // 16):
        vals_v.at[0, r, pl.ds(k * 16, 16)][...] = jnp.zeros((16,), jnp.float32)

    row0 = s * ROWS_PER_TILE
    nfull, rem = divmod(ROWS_PER_TILE, CHUNK)
    @pl.loop(0, nfull)
    def _(i):
      pltpu.sync_copy(vals_v.at[0], agg_s.at[pl.ds(row0 + i * CHUNK, CHUNK)])
    if rem:
      pltpu.sync_copy(vals_v.at[0].at[pl.ds(0, rem)],
                      agg_s.at[pl.ds(row0 + nfull * CHUNK, rem)])

    @pl.loop(0, nfull)
    def _(i):
      r = row0 + i * CHUNK
      pltpu.sync_copy(y_hbm.at[pl.ds(r, CHUNK)], y_s.at[pl.ds(r, CHUNK)])
    if rem:
      r = row0 + nfull * CHUNK
      pltpu.sync_copy(y_hbm.at[pl.ds(r, rem)], y_s.at[pl.ds(r, rem)])

    pltpu.sync_copy(src_hbm.at[wid], src_v)
    pltpu.sync_copy(dst_hbm.at[wid], dst_v)

    plsc.subcore_barrier()

    def start_gather(j, b):
      pltpu.async_copy(y_s.at[src_v.at[j]], vals_v.at[b], gsem[b])

    def wait_gather(j, b):
      pltpu.make_async_copy(y_s.at[src_v.at[j]], vals_v.at[b],
                            gsem[b]).wait()

    def start_scatter(j, b):
      pltpu.async_copy(vals_v.at[b], agg_s.at[dst_v.at[j]], ssem[b],
                       add=True)

    def wait_scatter(j, b):
      pltpu.make_async_copy(vals_v.at[b], agg_s.at[dst_v.at[j]],
                            ssem[b]).wait()

    for b in range(NBUF):
      start_gather(b, b)

    @pl.loop(0, NCHUNK - NBUF, step=NBUF)
    def _(j):
      for b in range(NBUF):
        wait_gather(j + b, b)
        start_scatter(j + b, b)
      for b in range(NBUF):
        wait_scatter(j + b, b)
        start_gather(j + b + NBUF, b)

    last = NCHUNK - NBUF
    for b in range(NBUF):
      wait_gather(last + b, b)
      start_scatter(last + b, b)
    for b in range(NBUF):
      wait_scatter(last + b, b)

    plsc.subcore_barrier()

    @pl.loop(0, nfull)
    def _(i):
      r = row0 + i * CHUNK
      pltpu.sync_copy(agg_s.at[pl.ds(r, CHUNK)],
                      out_hbm.at[c].at[pl.ds(r, CHUNK)])
    if rem:
      r = row0 + nfull * CHUNK
      pltpu.sync_copy(agg_s.at[pl.ds(r, rem)],
                      out_hbm.at[c].at[pl.ds(r, rem)])

  return edge_pass


_edge_pass_l1 = _make_edge_pass(H)
_edge_pass_l2 = _make_edge_pass(W2P)


@functools.partial(
    pl.kernel,
    out_type=jax.ShapeDtypeStruct((NC, NP), jnp.float32),
    mesh=_mesh,
    compiler_params=pltpu.CompilerParams(use_tc_tiling_on_sc=False),
    scratch_types=[
        pltpu.VMEM((NCHUNK, CHUNK), jnp.int32),
        pltpu.VMEM((CHUNK,), jnp.float32),
        pltpu.VMEM((ROWS_PER_TILE,), jnp.float32),
        pltpu.VMEM_SHARED((NP,), jnp.float32),
    ],
)
def _deg_pass(dst_hbm, out_hbm, dst_v, ones_v, zeros_v, deg_s):
  c = lax.axis_index("c")
  s = lax.axis_index("s")
  wid = c * NS + s

  for k in range(CHUNK // 16):
    ones_v.at[pl.ds(k * 16, 16)][...] = jnp.ones((16,), jnp.float32)

  @pl.loop(0, ROWS_PER_TILE // 16)
  def _(k):
    zeros_v.at[pl.ds(k * 16, 16)][...] = jnp.zeros((16,), jnp.float32)

  row0 = s * ROWS_PER_TILE
  pltpu.sync_copy(zeros_v, deg_s.at[pl.ds(row0, ROWS_PER_TILE)])
  pltpu.sync_copy(dst_hbm.at[wid], dst_v)

  plsc.subcore_barrier()

  @pl.loop(0, NCHUNK)
  def _(j):
    pltpu.sync_copy(ones_v, deg_s.at[dst_v.at[j]], add=True)

  plsc.subcore_barrier()

  pltpu.sync_copy(deg_s.at[pl.ds(row0, ROWS_PER_TILE)],
                  out_hbm.at[c].at[pl.ds(row0, ROWS_PER_TILE)])


def _tc_xw(x_ref, w_ref, o_ref):
  o_ref[...] = jnp.dot(x_ref[...], w_ref[...],
                       preferred_element_type=jnp.float32)


def _tc_scale(degp_ref, xw_ref, dis_ref, y_ref):
  deg = degp_ref[0] + degp_ref[1] + 1.0
  dis = lax.rsqrt(deg)[:, None]
  dis_ref[...] = dis
  y_ref[...] = xw_ref[...] * dis


def _tc_mid(p_ref, y_ref, dis_ref, b1_ref, w2_ref, z_ref):
  dis = dis_ref[...]
  h = dis * (p_ref[0] + p_ref[1] + y_ref[...]) + b1_ref[...]
  h = jnp.maximum(h, 0.0)
  z_ref[...] = dis * jnp.dot(h, w2_ref[...],
                             preferred_element_type=jnp.float32)


def _tc_final(q_ref, z_ref, dis_ref, b2_ref, o_ref):
  out = dis_ref[...] * (q_ref[0] + q_ref[1] + z_ref[...])
  o_ref[...] = out[:N, :2] + b2_ref[...]


def kernel(x, edge_index, W1, b1, W2, b2):
  ei = edge_index.astype(jnp.int32)
  pad = jnp.full((EPAD - E,), N, jnp.int32)
  srcp = jnp.concatenate([ei[0], pad]).reshape(NT, NCHUNK, CHUNK)
  dstp = jnp.concatenate([ei[1], pad]).reshape(NT, NCHUNK, CHUNK)
  x_pad = jnp.pad(x, ((0, NP - N), (0, 0)))
  w2_pad = jnp.pad(W2, ((0, 0), (0, W2P - 2)))

  degp = _deg_pass(dstp)

  xw = pl.pallas_call(
      _tc_xw,
      out_shape=jax.ShapeDtypeStruct((NP, H), jnp.float32),
  )(x_pad, W1)

  dis, y = pl.pallas_call(
      _tc_scale,
      out_shape=(jax.ShapeDtypeStruct((NP, 1), jnp.float32),
                 jax.ShapeDtypeStruct((NP, H), jnp.float32)),
  )(degp, xw)

  p = _edge_pass_l1(srcp, dstp, y)

  z = pl.pallas_call(
      _tc_mid,
      out_shape=jax.ShapeDtypeStruct((NP, W2P), jnp.float32),
  )(p, y, dis, b1.reshape(1, H), w2_pad)

  q = _edge_pass_l2(srcp, dstp, z)

  out = pl.pallas_call(
      _tc_final,
      out_shape=jax.ShapeDtypeStruct((N, 2), jnp.float32),
  )(q, z, dis, b2.reshape(1, 2))

  return out


# speedup vs baseline: 1.7594x; 1.1327x over previous
import functools

import jax
import jax.numpy as jnp
from jax import lax
from jax.experimental import pallas as pl
from jax.experimental.pallas import tpu as pltpu
from jax.experimental.pallas import tpu_sc as plsc

N = 10000
E = 320000
D_IN = 128
H = 32
W2P = 16

NC, NS = 2, 16
NT = NC * NS
CHUNK = 400
NCHUNK = 25
NBUF = 5
PER_TILE = NCHUNK * CHUNK
ROWS_PER_TILE = N // NS

_mesh = plsc.VectorSubcoreMesh(core_axis_name="c", subcore_axis_name="s")
_sc_params = pltpu.CompilerParams(use_tc_tiling_on_sc=False)


def _make_edge_pass(width):
  @functools.partial(
      pl.kernel,
      out_type=jax.ShapeDtypeStruct((NC, N, width), jnp.float32),
      mesh=_mesh,
      compiler_params=_sc_params,
      scratch_types=[
          pltpu.VMEM((NCHUNK, CHUNK), jnp.int32),
          pltpu.VMEM((NCHUNK, CHUNK), jnp.int32),
          pltpu.VMEM((NBUF, CHUNK, width), jnp.float32),
          pltpu.VMEM_SHARED((N, width), jnp.float32),
          pltpu.VMEM_SHARED((N, width), jnp.float32),
          [pltpu.SemaphoreType.DMA] * NBUF,
          [pltpu.SemaphoreType.DMA] * NBUF,
      ],
  )
  def edge_pass(src_hbm, dst_hbm, y_hbm, out_hbm,
                src_v, dst_v, vals_v, agg_s, y_s, gsem, ssem):
    c = lax.axis_index("c")
    s = lax.axis_index("s")
    wid = c * NS + s

    @pl.loop(0, CHUNK)
    def _(r):
      for k in range(width // 16):
        vals_v.at[0, r, pl.ds(k * 16, 16)][...] = jnp.zeros((16,), jnp.float32)

    row0 = s * ROWS_PER_TILE
    nfull, rem = divmod(ROWS_PER_TILE, CHUNK)

    def over_rows(fn):
      @pl.loop(0, nfull)
      def _(i):
        fn(row0 + i * CHUNK, CHUNK)
      if rem:
        fn(row0 + nfull * CHUNK, rem)

    over_rows(lambda r, n: pltpu.sync_copy(
        vals_v.at[0].at[pl.ds(0, n)], agg_s.at[pl.ds(r, n)]))
    over_rows(lambda r, n: pltpu.sync_copy(
        y_hbm.at[pl.ds(r, n)], y_s.at[pl.ds(r, n)]))

    pltpu.sync_copy(src_hbm.at[wid], src_v)
    pltpu.sync_copy(dst_hbm.at[wid], dst_v)

    plsc.subcore_barrier()

    def start_gather(j, b):
      pltpu.async_copy(y_s.at[src_v.at[j]], vals_v.at[b], gsem[b])

    def wait_gather(j, b):
      pltpu.make_async_copy(y_s.at[src_v.at[j]], vals_v.at[b],
                            gsem[b]).wait()

    def start_scatter(j, b):
      pltpu.async_copy(vals_v.at[b], agg_s.at[dst_v.at[j]], ssem[b],
                       add=True)

    def wait_scatter(j, b):
      pltpu.make_async_copy(vals_v.at[b], agg_s.at[dst_v.at[j]],
                            ssem[b]).wait()

    for b in range(NBUF):
      start_gather(b, b)

    @pl.loop(0, NCHUNK - NBUF, step=NBUF)
    def _(j):
      for b in range(NBUF):
        wait_gather(j + b, b)
        start_scatter(j + b, b)
      for b in range(NBUF):
        wait_scatter(j + b, b)
        start_gather(j + b + NBUF, b)

    last = NCHUNK - NBUF
    for b in range(NBUF):
      wait_gather(last + b, b)
      start_scatter(last + b, b)
    for b in range(NBUF):
      wait_scatter(last + b, b)

    plsc.subcore_barrier()

    over_rows(lambda r, n: pltpu.sync_copy(
        agg_s.at[pl.ds(r, n)], out_hbm.at[c].at[pl.ds(r, n)]))

  return edge_pass


_edge_pass_l1 = _make_edge_pass(H)
_edge_pass_l2 = _make_edge_pass(W2P)

DEG_ROWS = 624


@functools.partial(
    pl.kernel,
    out_type=jax.ShapeDtypeStruct((NC, N), jnp.float32),
    mesh=_mesh,
    compiler_params=_sc_params,
    scratch_types=[
        pltpu.VMEM((NCHUNK, CHUNK), jnp.int32),
        pltpu.VMEM((CHUNK,), jnp.float32),
        pltpu.VMEM((640,), jnp.float32),
        pltpu.VMEM_SHARED((N,), jnp.float32),
        pltpu.SemaphoreType.DMA,
    ],
)
def _deg_pass(dst_hbm, out_hbm, dst_v, ones_v, zeros_v, deg_s, sem):
  c = lax.axis_index("c")
  s = lax.axis_index("s")
  wid = c * NS + s

  for k in range(CHUNK // 16):
    ones_v.at[pl.ds(k * 16, 16)][...] = jnp.ones((16,), jnp.float32)

  @pl.loop(0, 640 // 16)
  def _(k):
    zeros_v.at[pl.ds(k * 16, 16)][...] = jnp.zeros((16,), jnp.float32)

  row0 = s * DEG_ROWS

  @pl.when(s < NS - 1)
  def _():
    pltpu.sync_copy(zeros_v.at[pl.ds(0, DEG_ROWS)],
                    deg_s.at[pl.ds(row0, DEG_ROWS)])

  @pl.when(s == NS - 1)
  def _():
    pltpu.sync_copy(zeros_v, deg_s.at[pl.ds((NS - 1) * DEG_ROWS, 640)])

  pltpu.sync_copy(dst_hbm.at[wid], dst_v)

  plsc.subcore_barrier()

  @pl.loop(0, NCHUNK)
  def _(j):
    pltpu.async_copy(ones_v, deg_s.at[dst_v.at[j]], sem, add=True)

  @pl.loop(0, NCHUNK)
  def _(j):
    pltpu.make_async_copy(ones_v, deg_s.at[dst_v.at[j]], sem).wait()

  plsc.subcore_barrier()

  @pl.when(s < NS - 1)
  def _():
    pltpu.sync_copy(deg_s.at[pl.ds(row0, DEG_ROWS)],
                    out_hbm.at[c].at[pl.ds(row0, DEG_ROWS)])

  @pl.when(s == NS - 1)
  def _():
    pltpu.sync_copy(deg_s.at[pl.ds((NS - 1) * DEG_ROWS, 640)],
                    out_hbm.at[c].at[pl.ds((NS - 1) * DEG_ROWS, 640)])


def _tc_front(x_ref, w_ref, degp_ref, y_ref, dis_ref):
  deg = degp_ref[0] + degp_ref[1] + 1.0
  dis = lax.rsqrt(deg)[:, None]
  dis_ref[...] = dis
  y_ref[...] = dis * jnp.dot(x_ref[...], w_ref[...],
                             preferred_element_type=jnp.float32)


def _tc_mid(p_ref, y_ref, dis_ref, b1_ref, w2_ref, z_ref):
  dis = dis_ref[...]
  h = dis * (p_ref[0] + p_ref[1] + y_ref[...]) + b1_ref[...]
  h = jnp.maximum(h, 0.0)
  z_ref[...] = dis * jnp.dot(h, w2_ref[...],
                             preferred_element_type=jnp.float32)


def _tc_final(q_ref, z_ref, dis_ref, b2_ref, o_ref):
  out = dis_ref[...] * (q_ref[0] + q_ref[1] + z_ref[...])
  o_ref[...] = out[:, :2] + b2_ref[...]


def kernel(x, edge_index, W1, b1, W2, b2):
  ei = edge_index.astype(jnp.int32)
  srcp = ei[0].reshape(NT, NCHUNK, CHUNK)
  dstp = ei[1].reshape(NT, NCHUNK, CHUNK)
  w2_pad = jnp.pad(W2, ((0, 0), (0, W2P - 2)))

  degp = _deg_pass(dstp)

  y, dis = pl.pallas_call(
      _tc_front,
      out_shape=(jax.ShapeDtypeStruct((N, H), jnp.float32),
                 jax.ShapeDtypeStruct((N, 1), jnp.float32)),
  )(x, W1, degp)

  p = _edge_pass_l1(srcp, dstp, y)

  z = pl.pallas_call(
      _tc_mid,
      out_shape=jax.ShapeDtypeStruct((N, W2P), jnp.float32),
  )(p, y, dis, b1.reshape(1, H), w2_pad)

  q = _edge_pass_l2(srcp, dstp, z)

  out = pl.pallas_call(
      _tc_final,
      out_shape=jax.ShapeDtypeStruct((N, 2), jnp.float32),
  )(q, z, dis, b2.reshape(1, 2))

  return out


# speedup vs baseline: 1.8392x; 1.0454x over previous
import functools

import jax
import jax.numpy as jnp
from jax import lax
from jax.experimental import pallas as pl
from jax.experimental.pallas import tpu as pltpu
from jax.experimental.pallas import tpu_sc as plsc

N = 10000
E = 320000
D_IN = 128
H = 32
W2P = 16

NC, NS = 2, 16
NT = NC * NS
CHUNK = 400
NCHUNK = 25
NBUF = 5
PER_TILE = NCHUNK * CHUNK
ROWS_PER_TILE = N // NS

_mesh = plsc.VectorSubcoreMesh(core_axis_name="c", subcore_axis_name="s")
_sc_params = pltpu.CompilerParams(use_tc_tiling_on_sc=False)


def _make_edge_pass(width):
  @functools.partial(
      pl.kernel,
      out_type=jax.ShapeDtypeStruct((NC, N, width), jnp.float32),
      mesh=_mesh,
      compiler_params=_sc_params,
      scratch_types=[
          pltpu.VMEM((NCHUNK, CHUNK), jnp.int32),
          pltpu.VMEM((NCHUNK, CHUNK), jnp.int32),
          pltpu.VMEM((NBUF, CHUNK, width), jnp.float32),
          pltpu.VMEM_SHARED((N, width), jnp.float32),
          pltpu.VMEM_SHARED((N, width), jnp.float32),
          [pltpu.SemaphoreType.DMA] * NBUF,
          [pltpu.SemaphoreType.DMA] * NBUF,
      ],
  )
  def edge_pass(ei_hbm, y_hbm, zeros_hbm, out_hbm,
                src_v, dst_v, vals_v, agg_s, y_s, gsem, ssem):
    c = lax.axis_index("c")
    s = lax.axis_index("s")
    wid = c * NS + s
    y2 = y_hbm
    out2 = out_hbm.at[c]

    pltpu.sync_copy(zeros_hbm, vals_v.at[0])

    row0 = s * ROWS_PER_TILE
    nfull, rem = divmod(ROWS_PER_TILE, CHUNK)

    def over_rows(fn):
      @pl.loop(0, nfull)
      def _(i):
        fn(row0 + i * CHUNK, CHUNK)
      if rem:
        fn(row0 + nfull * CHUNK, rem)

    over_rows(lambda r, n: pltpu.sync_copy(
        vals_v.at[0].at[pl.ds(0, n)], agg_s.at[pl.ds(r, n)]))
    over_rows(lambda r, n: pltpu.sync_copy(
        y2.at[pl.ds(r, n)], y_s.at[pl.ds(r, n)]))

    pltpu.sync_copy(ei_hbm.at[0, wid], src_v)
    pltpu.sync_copy(ei_hbm.at[1, wid], dst_v)

    plsc.subcore_barrier()

    def start_gather(j, b):
      pltpu.async_copy(y_s.at[src_v.at[j]], vals_v.at[b], gsem[b])

    def wait_gather(j, b):
      pltpu.make_async_copy(y_s.at[src_v.at[j]], vals_v.at[b],
                            gsem[b]).wait()

    def start_scatter(j, b):
      pltpu.async_copy(vals_v.at[b], agg_s.at[dst_v.at[j]], ssem[b],
                       add=True)

    def wait_scatter(j, b):
      pltpu.make_async_copy(vals_v.at[b], agg_s.at[dst_v.at[j]],
                            ssem[b]).wait()

    for b in range(NBUF):
      start_gather(b, b)

    @pl.loop(0, NCHUNK - NBUF, step=NBUF)
    def _(j):
      for b in range(NBUF):
        wait_gather(j + b, b)
        start_scatter(j + b, b)
      for b in range(NBUF):
        wait_scatter(j + b, b)
        start_gather(j + b + NBUF, b)

    last = NCHUNK - NBUF
    for b in range(NBUF):
      wait_gather(last + b, b)
      start_scatter(last + b, b)
    for b in range(NBUF):
      wait_scatter(last + b, b)

    plsc.subcore_barrier()

    over_rows(lambda r, n: pltpu.sync_copy(
        agg_s.at[pl.ds(r, n)], out2.at[pl.ds(r, n)]))

  return edge_pass


_edge_pass_l1 = _make_edge_pass(H)
_edge_pass_l2 = _make_edge_pass(W2P)

DEG_ROWS = 624


@functools.partial(
    pl.kernel,
    out_type=jax.ShapeDtypeStruct((NC, N), jnp.float32),
    mesh=_mesh,
    compiler_params=_sc_params,
    scratch_types=[
        pltpu.VMEM((NCHUNK, CHUNK), jnp.int32),
        pltpu.VMEM((CHUNK,), jnp.float32),
        pltpu.VMEM((640,), jnp.float32),
        pltpu.VMEM_SHARED((N,), jnp.float32),
        pltpu.SemaphoreType.DMA,
    ],
)
def _deg_pass(ei_hbm, out_hbm, dst_v, ones_v, zeros_v, deg_s, sem):
  c = lax.axis_index("c")
  s = lax.axis_index("s")
  wid = c * NS + s

  for k in range(CHUNK // 16):
    ones_v.at[pl.ds(k * 16, 16)][...] = jnp.ones((16,), jnp.float32)

  @pl.loop(0, 640 // 16)
  def _(k):
    zeros_v.at[pl.ds(k * 16, 16)][...] = jnp.zeros((16,), jnp.float32)

  row0 = s * DEG_ROWS

  @pl.when(s < NS - 1)
  def _():
    pltpu.sync_copy(zeros_v.at[pl.ds(0, DEG_ROWS)],
                    deg_s.at[pl.ds(row0, DEG_ROWS)])

  @pl.when(s == NS - 1)
  def _():
    pltpu.sync_copy(zeros_v, deg_s.at[pl.ds((NS - 1) * DEG_ROWS, 640)])

  pltpu.sync_copy(ei_hbm.at[1, wid], dst_v)

  plsc.subcore_barrier()

  @pl.loop(0, NCHUNK)
  def _(j):
    pltpu.async_copy(ones_v, deg_s.at[dst_v.at[j]], sem, add=True)

  @pl.loop(0, NCHUNK)
  def _(j):
    pltpu.make_async_copy(ones_v, deg_s.at[dst_v.at[j]], sem).wait()

  plsc.subcore_barrier()

  @pl.when(s < NS - 1)
  def _():
    pltpu.sync_copy(deg_s.at[pl.ds(row0, DEG_ROWS)],
                    out_hbm.at[c].at[pl.ds(row0, DEG_ROWS)])

  @pl.when(s == NS - 1)
  def _():
    pltpu.sync_copy(deg_s.at[pl.ds((NS - 1) * DEG_ROWS, 640)],
                    out_hbm.at[c].at[pl.ds((NS - 1) * DEG_ROWS, 640)])


def _tc_front(x_ref, w_ref, degp_ref, y_ref, dis_ref):
  deg = degp_ref[0] + degp_ref[1] + 1.0
  dis = lax.rsqrt(deg)[:, None]
  dis_ref[...] = dis
  y_ref[...] = dis * jnp.dot(x_ref[...], w_ref[...],
                             preferred_element_type=jnp.float32)


def _tc_mid(p_ref, y_ref, dis_ref, b1_ref, w2_ref, z_ref):
  dis = dis_ref[...]
  h = dis * (p_ref[0] + p_ref[1] + y_ref[...]) + b1_ref[...]
  h = jnp.maximum(h, 0.0)
  z_ref[...] = dis * jnp.dot(h, w2_ref[...],
                             preferred_element_type=jnp.float32)


def _tc_final(q_ref, z_ref, dis_ref, b2_ref, o_ref):
  out = dis_ref[...] * (q_ref[0] + q_ref[1] + z_ref[...])
  o_ref[...] = out[:, :2] + b2_ref[...]


def kernel(x, edge_index, W1, b1, W2, b2):
  eir = edge_index.astype(jnp.int32).reshape(2, NT, NCHUNK, CHUNK)

  degp = _deg_pass(eir)

  y, dis = pl.pallas_call(
      _tc_front,
      out_shape=(jax.ShapeDtypeStruct((N, H), jnp.float32),
                 jax.ShapeDtypeStruct((N, 1), jnp.float32)),
  )(x, W1, degp)

  zeros_h = jnp.zeros((CHUNK, H), jnp.float32)
  zeros_2 = jnp.zeros((CHUNK, W2P), jnp.float32)
  p = _edge_pass_l1(eir, y, zeros_h)

  z = pl.pallas_call(
      _tc_mid,
      out_shape=jax.ShapeDtypeStruct((N, W2P), jnp.float32),
  )(p, y, dis, b1.reshape(1, H), jnp.pad(W2, ((0, 0), (0, W2P - 2))))

  q = _edge_pass_l2(eir, z, zeros_2)

  out = pl.pallas_call(
      _tc_final,
      out_shape=jax.ShapeDtypeStruct((N, 2), jnp.float32),
  )(q, z, dis, b2.reshape(1, 2))

  return out


# speedup vs baseline: 1.9175x; 1.0426x over previous
import functools

import jax
import jax.numpy as jnp
from jax import lax
from jax.experimental import pallas as pl
from jax.experimental.pallas import tpu as pltpu
from jax.experimental.pallas import tpu_sc as plsc

N = 10000
E = 320000
D_IN = 128
H = 32
W2P = 8

NC, NS = 2, 16
NT = NC * NS
CHUNK = 400
NCHUNK = 25
NBUF = 5
PER_TILE = NCHUNK * CHUNK
ROWS_PER_TILE = N // NS

_mesh = plsc.VectorSubcoreMesh(core_axis_name="c", subcore_axis_name="s")
_sc_params = pltpu.CompilerParams(use_tc_tiling_on_sc=False)


def _make_edge_pass(width):
  @functools.partial(
      pl.kernel,
      out_type=jax.ShapeDtypeStruct((NC, N, width), jnp.float32),
      mesh=_mesh,
      compiler_params=_sc_params,
      scratch_types=[
          pltpu.VMEM((NCHUNK, CHUNK), jnp.int32),
          pltpu.VMEM((NCHUNK, CHUNK), jnp.int32),
          pltpu.VMEM((NBUF, CHUNK, width), jnp.float32),
          pltpu.VMEM_SHARED((N, width), jnp.float32),
          pltpu.VMEM_SHARED((N, width), jnp.float32),
          [pltpu.SemaphoreType.DMA] * NBUF,
          [pltpu.SemaphoreType.DMA] * NBUF,
      ],
  )
  def edge_pass(ei_hbm, y_hbm, zeros_hbm, out_hbm,
                src_v, dst_v, vals_v, agg_s, y_s, gsem, ssem):
    c = lax.axis_index("c")
    s = lax.axis_index("s")
    wid = c * NS + s
    y2 = y_hbm
    out2 = out_hbm.at[c]

    pltpu.sync_copy(zeros_hbm, vals_v.at[0])

    row0 = s * ROWS_PER_TILE
    nfull, rem = divmod(ROWS_PER_TILE, CHUNK)

    def over_rows(fn):
      @pl.loop(0, nfull)
      def _(i):
        fn(row0 + i * CHUNK, CHUNK)
      if rem:
        fn(row0 + nfull * CHUNK, rem)

    over_rows(lambda r, n: pltpu.sync_copy(
        vals_v.at[0].at[pl.ds(0, n)], agg_s.at[pl.ds(r, n)]))
    over_rows(lambda r, n: pltpu.sync_copy(
        y2.at[pl.ds(r, n)], y_s.at[pl.ds(r, n)]))

    pltpu.sync_copy(ei_hbm.at[0, wid], src_v)
    pltpu.sync_copy(ei_hbm.at[1, wid], dst_v)

    plsc.subcore_barrier()

    def start_gather(j, b):
      pltpu.async_copy(y_s.at[src_v.at[j]], vals_v.at[b], gsem[b])

    def wait_gather(j, b):
      pltpu.make_async_copy(y_s.at[src_v.at[j]], vals_v.at[b],
                            gsem[b]).wait()

    def start_scatter(j, b):
      pltpu.async_copy(vals_v.at[b], agg_s.at[dst_v.at[j]], ssem[b],
                       add=True)

    def wait_scatter(j, b):
      pltpu.make_async_copy(vals_v.at[b], agg_s.at[dst_v.at[j]],
                            ssem[b]).wait()

    for b in range(NBUF):
      start_gather(b, b)

    @pl.loop(0, NCHUNK - NBUF, step=NBUF)
    def _(j):
      for b in range(NBUF):
        wait_gather(j + b, b)
        start_scatter(j + b, b)
      for b in range(NBUF):
        wait_scatter(j + b, b)
        start_gather(j + b + NBUF, b)

    last = NCHUNK - NBUF
    for b in range(NBUF):
      wait_gather(last + b, b)
      start_scatter(last + b, b)
    for b in range(NBUF):
      wait_scatter(last + b, b)

    plsc.subcore_barrier()

    over_rows(lambda r, n: pltpu.sync_copy(
        agg_s.at[pl.ds(r, n)], out2.at[pl.ds(r, n)]))

  return edge_pass


_edge_pass_l1 = _make_edge_pass(H)
_edge_pass_l2 = _make_edge_pass(W2P)

DEG_ROWS = 624


@functools.partial(
    pl.kernel,
    out_type=jax.ShapeDtypeStruct((NC, N), jnp.float32),
    mesh=_mesh,
    compiler_params=_sc_params,
    scratch_types=[
        pltpu.VMEM((NCHUNK, CHUNK), jnp.int32),
        pltpu.VMEM((CHUNK,), jnp.float32),
        pltpu.VMEM((640,), jnp.float32),
        pltpu.VMEM_SHARED((N,), jnp.float32),
        pltpu.SemaphoreType.DMA,
    ],
)
def _deg_pass(ei_hbm, out_hbm, dst_v, ones_v, zeros_v, deg_s, sem):
  c = lax.axis_index("c")
  s = lax.axis_index("s")
  wid = c * NS + s

  for k in range(CHUNK // 16):
    ones_v.at[pl.ds(k * 16, 16)][...] = jnp.ones((16,), jnp.float32)

  @pl.loop(0, 640 // 16)
  def _(k):
    zeros_v.at[pl.ds(k * 16, 16)][...] = jnp.zeros((16,), jnp.float32)

  row0 = s * DEG_ROWS

  @pl.when(s < NS - 1)
  def _():
    pltpu.sync_copy(zeros_v.at[pl.ds(0, DEG_ROWS)],
                    deg_s.at[pl.ds(row0, DEG_ROWS)])

  @pl.when(s == NS - 1)
  def _():
    pltpu.sync_copy(zeros_v, deg_s.at[pl.ds((NS - 1) * DEG_ROWS, 640)])

  pltpu.sync_copy(ei_hbm.at[1, wid], dst_v)

  plsc.subcore_barrier()

  @pl.loop(0, NCHUNK)
  def _(j):
    pltpu.async_copy(ones_v, deg_s.at[dst_v.at[j]], sem, add=True)

  @pl.loop(0, NCHUNK)
  def _(j):
    pltpu.make_async_copy(ones_v, deg_s.at[dst_v.at[j]], sem).wait()

  plsc.subcore_barrier()

  @pl.when(s < NS - 1)
  def _():
    pltpu.sync_copy(deg_s.at[pl.ds(row0, DEG_ROWS)],
                    out_hbm.at[c].at[pl.ds(row0, DEG_ROWS)])

  @pl.when(s == NS - 1)
  def _():
    pltpu.sync_copy(deg_s.at[pl.ds((NS - 1) * DEG_ROWS, 640)],
                    out_hbm.at[c].at[pl.ds((NS - 1) * DEG_ROWS, 640)])


def _tc_front(x_ref, w_ref, degp_ref, y_ref, dis_ref):
  deg = degp_ref[0] + degp_ref[1] + 1.0
  dis = lax.rsqrt(deg)[:, None]
  dis_ref[...] = dis
  y_ref[...] = dis * jnp.dot(x_ref[...], w_ref[...],
                             preferred_element_type=jnp.float32)


def _tc_mid(p_ref, y_ref, dis_ref, b1_ref, w2_ref, z_ref):
  dis = dis_ref[...]
  h = dis * (p_ref[0] + p_ref[1] + y_ref[...]) + b1_ref[...]
  h = jnp.maximum(h, 0.0)
  z_ref[...] = dis * jnp.dot(h, w2_ref[...],
                             preferred_element_type=jnp.float32)


def _tc_final(q_ref, z_ref, dis_ref, b2_ref, o_ref):
  out = dis_ref[...] * (q_ref[0] + q_ref[1] + z_ref[...])
  o_ref[...] = out[:, :2] + b2_ref[...]


def kernel(x, edge_index, W1, b1, W2, b2):
  eir = edge_index.astype(jnp.int32).reshape(2, NT, NCHUNK, CHUNK)

  degp = _deg_pass(eir)

  y, dis = pl.pallas_call(
      _tc_front,
      out_shape=(jax.ShapeDtypeStruct((N, H), jnp.float32),
                 jax.ShapeDtypeStruct((N, 1), jnp.float32)),
  )(x, W1, degp)

  zeros_h = jnp.zeros((CHUNK, H), jnp.float32)
  zeros_2 = jnp.zeros((CHUNK, W2P), jnp.float32)
  p = _edge_pass_l1(eir, y, zeros_h)

  z = pl.pallas_call(
      _tc_mid,
      out_shape=jax.ShapeDtypeStruct((N, W2P), jnp.float32),
  )(p, y, dis, b1.reshape(1, H), jnp.pad(W2, ((0, 0), (0, W2P - 2))))

  q = _edge_pass_l2(eir, z, zeros_2)

  out = pl.pallas_call(
      _tc_final,
      out_shape=jax.ShapeDtypeStruct((N, 2), jnp.float32),
  )(q, z, dis, b2.reshape(1, 2))

  return out


# speedup vs baseline: 1.9678x; 1.0262x over previous
import functools

import jax
import jax.numpy as jnp
from jax import lax
from jax.experimental import pallas as pl
from jax.experimental.pallas import tpu as pltpu
from jax.experimental.pallas import tpu_sc as plsc

N = 10000
E = 320000
D_IN = 128
H = 32
W2P = 8

NC, NS = 2, 16
NT = NC * NS
CHUNK = 400
NCHUNK = 25
NBUF = 5
PER_TILE = NCHUNK * CHUNK
ROWS_PER_TILE = N // NS

_mesh = plsc.VectorSubcoreMesh(core_axis_name="c", subcore_axis_name="s")
_sc_params = pltpu.CompilerParams(use_tc_tiling_on_sc=False)


def _make_edge_pass(width):
  @functools.partial(
      pl.kernel,
      out_type=jax.ShapeDtypeStruct((NC, N, width), jnp.float32),
      mesh=_mesh,
      compiler_params=_sc_params,
      scratch_types=[
          pltpu.VMEM((NCHUNK, CHUNK), jnp.int32),
          pltpu.VMEM((NCHUNK, CHUNK), jnp.int32),
          pltpu.VMEM((NBUF, CHUNK, width), jnp.float32),
          pltpu.VMEM_SHARED((N, width), jnp.float32),
          pltpu.VMEM_SHARED((N, width), jnp.float32),
          [pltpu.SemaphoreType.DMA] * NBUF,
          [pltpu.SemaphoreType.DMA] * NBUF,
      ],
  )
  def edge_pass(ei_hbm, y_hbm, zeros_hbm, out_hbm,
                src_v, dst_v, vals_v, agg_s, y_s, gsem, ssem):
    c = lax.axis_index("c")
    s = lax.axis_index("s")
    wid = c * NS + s
    out2 = out_hbm.at[c]

    pltpu.sync_copy(zeros_hbm, vals_v.at[0])

    row0 = s * ROWS_PER_TILE
    nfull, rem = divmod(ROWS_PER_TILE, CHUNK)

    def over_rows(fn):
      @pl.loop(0, nfull)
      def _(i):
        fn(row0 + i * CHUNK, CHUNK)
      if rem:
        fn(row0 + nfull * CHUNK, rem)

    over_rows(lambda r, n: pltpu.sync_copy(
        vals_v.at[0].at[pl.ds(0, n)], agg_s.at[pl.ds(r, n)]))
    over_rows(lambda r, n: pltpu.sync_copy(
        y_hbm.at[pl.ds(r, n)], y_s.at[pl.ds(r, n)]))

    pltpu.sync_copy(ei_hbm.at[0, wid], src_v)
    pltpu.sync_copy(ei_hbm.at[1, wid], dst_v)

    plsc.subcore_barrier()

    def gather_src(b):
      return y_hbm if b >= 4 else y_s

    def start_gather(j, b):
      pltpu.async_copy(gather_src(b).at[src_v.at[j]], vals_v.at[b], gsem[b])

    def wait_gather(j, b):
      pltpu.make_async_copy(gather_src(b).at[src_v.at[j]], vals_v.at[b],
                            gsem[b]).wait()

    def start_scatter(j, b):
      pltpu.async_copy(vals_v.at[b], agg_s.at[dst_v.at[j]], ssem[b],
                       add=True)

    def wait_scatter(j, b):
      pltpu.make_async_copy(vals_v.at[b], agg_s.at[dst_v.at[j]],
                            ssem[b]).wait()

    for b in range(NBUF):
      start_gather(b, b)

    @pl.loop(0, NCHUNK - NBUF, step=NBUF)
    def _(j):
      for b in range(NBUF):
        wait_gather(j + b, b)
        start_scatter(j + b, b)
      for b in range(NBUF):
        wait_scatter(j + b, b)
        start_gather(j + b + NBUF, b)

    last = NCHUNK - NBUF
    for b in range(NBUF):
      wait_gather(last + b, b)
      start_scatter(last + b, b)
    for b in range(NBUF):
      wait_scatter(last + b, b)

    plsc.subcore_barrier()

    over_rows(lambda r, n: pltpu.sync_copy(
        agg_s.at[pl.ds(r, n)], out2.at[pl.ds(r, n)]))

  return edge_pass


_edge_pass_l1 = _make_edge_pass(H)
_edge_pass_l2 = _make_edge_pass(W2P)

DEG_ROWS = 624


@functools.partial(
    pl.kernel,
    out_type=jax.ShapeDtypeStruct((NC, N), jnp.float32),
    mesh=_mesh,
    compiler_params=_sc_params,
    scratch_types=[
        pltpu.VMEM((NCHUNK, CHUNK), jnp.int32),
        pltpu.VMEM((CHUNK,), jnp.float32),
        pltpu.VMEM((640,), jnp.float32),
        pltpu.VMEM_SHARED((N,), jnp.float32),
        pltpu.SemaphoreType.DMA,
    ],
)
def _deg_pass(ei_hbm, out_hbm, dst_v, ones_v, zeros_v, deg_s, sem):
  c = lax.axis_index("c")
  s = lax.axis_index("s")
  wid = c * NS + s

  for k in range(CHUNK // 16):
    ones_v.at[pl.ds(k * 16, 16)][...] = jnp.ones((16,), jnp.float32)

  @pl.loop(0, 640 // 16)
  def _(k):
    zeros_v.at[pl.ds(k * 16, 16)][...] = jnp.zeros((16,), jnp.float32)

  row0 = s * DEG_ROWS

  @pl.when(s < NS - 1)
  def _():
    pltpu.sync_copy(zeros_v.at[pl.ds(0, DEG_ROWS)],
                    deg_s.at[pl.ds(row0, DEG_ROWS)])

  @pl.when(s == NS - 1)
  def _():
    pltpu.sync_copy(zeros_v, deg_s.at[pl.ds((NS - 1) * DEG_ROWS, 640)])

  pltpu.sync_copy(ei_hbm.at[1, wid], dst_v)

  plsc.subcore_barrier()

  @pl.loop(0, NCHUNK)
  def _(j):
    pltpu.async_copy(ones_v, deg_s.at[dst_v.at[j]], sem, add=True)

  @pl.loop(0, NCHUNK)
  def _(j):
    pltpu.make_async_copy(ones_v, deg_s.at[dst_v.at[j]], sem).wait()

  plsc.subcore_barrier()

  @pl.when(s < NS - 1)
  def _():
    pltpu.sync_copy(deg_s.at[pl.ds(row0, DEG_ROWS)],
                    out_hbm.at[c].at[pl.ds(row0, DEG_ROWS)])

  @pl.when(s == NS - 1)
  def _():
    pltpu.sync_copy(deg_s.at[pl.ds((NS - 1) * DEG_ROWS, 640)],
                    out_hbm.at[c].at[pl.ds((NS - 1) * DEG_ROWS, 640)])


def _tc_front(x_ref, w_ref, degp_ref, y_ref, dis_ref):
  deg = degp_ref[0] + degp_ref[1] + 1.0
  dis = lax.rsqrt(deg)[:, None]
  dis_ref[...] = dis
  y_ref[...] = dis * jnp.dot(x_ref[...], w_ref[...],
                             preferred_element_type=jnp.float32)


def _tc_mid(p_ref, y_ref, dis_ref, b1_ref, w2_ref, z_ref):
  dis = dis_ref[...]
  h = dis * (p_ref[0] + p_ref[1] + y_ref[...]) + b1_ref[...]
  h = jnp.maximum(h, 0.0)
  z_ref[...] = dis * jnp.dot(h, w2_ref[...],
                             preferred_element_type=jnp.float32)


def _tc_final(q_ref, z_ref, dis_ref, b2_ref, o_ref):
  out = dis_ref[...] * (q_ref[0] + q_ref[1] + z_ref[...])
  o_ref[...] = out[:, :2] + b2_ref[...]


def kernel(x, edge_index, W1, b1, W2, b2):
  eir = edge_index.astype(jnp.int32).reshape(2, NT, NCHUNK, CHUNK)

  degp = _deg_pass(eir)

  y, dis = pl.pallas_call(
      _tc_front,
      out_shape=(jax.ShapeDtypeStruct((N, H), jnp.float32),
                 jax.ShapeDtypeStruct((N, 1), jnp.float32)),
  )(x, W1, degp)

  zeros_h = jnp.zeros((CHUNK, H), jnp.float32)
  zeros_2 = jnp.zeros((CHUNK, W2P), jnp.float32)
  p = _edge_pass_l1(eir, y, zeros_h)

  z = pl.pallas_call(
      _tc_mid,
      out_shape=jax.ShapeDtypeStruct((N, W2P), jnp.float32),
  )(p, y, dis, b1.reshape(1, H), jnp.pad(W2, ((0, 0), (0, W2P - 2))))

  q = _edge_pass_l2(eir, z, zeros_2)

  out = pl.pallas_call(
      _tc_final,
      out_shape=jax.ShapeDtypeStruct((N, 2), jnp.float32),
  )(q, z, dis, b2.reshape(1, 2))

  return out
